# Initial kernel scaffold; baseline (speedup 1.0000x reference)
import jax
import jax.numpy as jnp
from jax.experimental import pallas as pl


def kernel(src_index, dst_index, timestamp, event_features, labels, time_w, time_b, gru_Wih, gru_Whh, gru_bih, gru_bhh, lin_W, lin_b):
    raise NotImplementedError("write your pallas kernel here")


# speedup vs baseline: 272.2002x; 272.2002x over previous
import functools

import jax
import jax.numpy as jnp
from jax import lax
from jax.experimental import pallas as pl
from jax.experimental.pallas import tpu as pltpu
from jax.experimental.pallas import tpu_sc as plsc

_N = 4096
_ROWS = 32
_LANES = 128
_CH = 1024
_NCH = _N // _CH
_MEM = 4
_NODES = 3


def _scan_body(t_ref, s_ref, d_ref, dts_ref, dtd_ref, sd_ref):
    t = t_ref[...]
    s = s_ref[...]
    d = d_ref[...]
    lane = lax.broadcasted_iota(jnp.int32, (_ROWS, _LANES), 1)
    row1 = lax.broadcasted_iota(jnp.int32, (_ROWS, 1), 0)
    prevs = []
    for node in range(_NODES):
        x = jnp.where((s == node) | (d == node), t, 0.0)
        for sh in (1, 2, 4, 8, 16, 32, 64):
            x = jnp.maximum(x, jnp.where(lane >= sh, jnp.roll(x, sh, axis=1), 0.0))
        rt = x[:, _LANES - 1:_LANES]
        e = jnp.where(row1 >= 1, jnp.roll(rt, 1, axis=0), 0.0)
        for sh in (1, 2, 4, 8, 16):
            e = jnp.maximum(e, jnp.where(row1 >= sh, jnp.roll(e, sh, axis=0), 0.0))
        incl = jnp.maximum(x, e)
        excl = jnp.where(lane >= 1, jnp.roll(incl, 1, axis=1), e)
        prevs.append(excl)
    ps = jnp.where(s == 0, prevs[0], jnp.where(s == 1, prevs[1], prevs[2]))
    pd = jnp.where(d == 0, prevs[0], jnp.where(d == 1, prevs[1], prevs[2]))
    dts_ref[...] = t - ps
    dtd_ref[...] = t - pd
    sd_ref[...] = s + d * 4


def _const_body(dts_ref, dtd_ref, raw_ref, w_ref, b_ref, m_ref, bias_ref, c_ref):
    dts = dts_ref[...]
    dtd = dtd_ref[...]
    raw = raw_ref[...]
    w = w_ref[...]
    b = b_ref[...]
    te_s = jnp.cos(dts * w + b)
    te_d = jnp.cos(dtd * w + b)
    acc = jnp.broadcast_to(bias_ref[...], (_N, 32))
    for k in range(2):
        acc = acc + raw[:, k:k + 1] * m_ref[k:k + 1, :]
    for k in range(4):
        acc = acc + te_s[:, k:k + 1] * m_ref[2 + k:3 + k, :]
    for k in range(4):
        acc = acc + te_d[:, k:k + 1] * m_ref[6 + k:7 + k, :]
    c_ref[...] = acc


def _logit_body(x8_ref, raw_ref, lm_ref, lr_ref, lb_ref, o_ref):
    x8 = x8_ref[...]
    raw = raw_ref[...]
    acc = jnp.broadcast_to(lb_ref[...], (_N, 2))
    for k in range(8):
        acc = acc + x8[:, k:k + 1] * lm_ref[k:k + 1, :]
    for k in range(2):
        acc = acc + raw[:, k:k + 1] * lr_ref[k:k + 1, :]
    o_ref[...] = acc


def _sc_chain_body(c_hbm, sd_hbm, w_hbm, out_hbm,
                   cbuf, sdbuf, wbuf, membuf, outbuf, sbuf, bbuf):
    cid = lax.axis_index("c")
    sid = lax.axis_index("s")

    @pl.when(jnp.logical_and(cid == 0, sid == 0))
    def _():
        pltpu.sync_copy(w_hbm, wbuf)
        membuf[...] = jnp.zeros((16,), jnp.float32)
        lane = lax.iota(jnp.int32, 16)
        lane_lt4 = lane < 4
        lane_mid = jnp.logical_and(lane >= 4, lane < 8)
        low2 = jnp.bitwise_and(lane, 3)
        perm_gh = jnp.where(lane < 8, lane + 8, lane)
        perm_r = jnp.where(lane_lt4, lane, jnp.where(lane < 8, lane + 4, lane))
        perm_z = jnp.where(lane_lt4, lane + 4, jnp.where(lane < 8, lane + 8, lane))
        wA = [plsc.load_gather(wbuf, [lane + 16 * k]) for k in range(8)]
        wB = [plsc.load_gather(wbuf, [lane + 16 * (8 + k)]) for k in range(8)]

        @pl.loop(0, _NCH)
        def _chunk(ci):
            pltpu.sync_copy(c_hbm.at[pl.ds(ci * (_CH * 32), _CH * 32)], cbuf)
            pltpu.sync_copy(sd_hbm.at[pl.ds(ci * _CH, _CH)], sdbuf)

            @pl.loop(0, _CH)
            def _step(j):
                jb = jnp.broadcast_to(j, (16,))
                sdv = plsc.load_gather(sdbuf, [jb])
                sv4 = jnp.bitwise_and(sdv, 3) * 4
                dv4 = lax.shift_right_logical(sdv, 2) * 4
                mem_idx = jnp.where(lane_lt4, sv4, dv4) + low2
                x8 = plsc.load_gather(membuf, [mem_idx])
                j32 = j * 32
                yA = plsc.load_gather(cbuf, [lane + j32])
                yB = plsc.load_gather(cbuf, [lane + (j32 + 16)])
                for k in range(8):
                    bidx = (sv4 + k) if k < 4 else (dv4 + (k - 4))
                    bk = plsc.load_gather(membuf, [bidx])
                    yA = yA + bk * wA[k]
                    yB = yB + bk * wB[k]
                S = 1.0 / (1.0 + jnp.exp(-yA))
                sbuf[...] = S
                bbuf[...] = yB
                gh8 = plsc.load_gather(bbuf, [perm_gh])
                r8 = plsc.load_gather(sbuf, [perm_r])
                z8 = plsc.load_gather(sbuf, [perm_z])
                nin = yB + r8 * gh8
                e2 = jnp.exp(-2.0 * nin)
                th = (1.0 - e2) / (1.0 + e2)
                new8 = th + z8 * (x8 - th)
                plsc.store_scatter(membuf, [mem_idx], new8, mask=lane_lt4)
                plsc.store_scatter(membuf, [mem_idx], new8, mask=lane_mid)
                plsc.store_scatter(outbuf, [lane + j * 16], x8)

            pltpu.sync_copy(outbuf, out_hbm.at[pl.ds(ci * (_CH * 16), _CH * 16)])


def _sc_chain(c_flat, sd, w16):
    mesh = plsc.VectorSubcoreMesh(core_axis_name="c", subcore_axis_name="s")
    f = functools.partial(
        pl.kernel,
        out_type=jax.ShapeDtypeStruct((_N * 16,), jnp.float32),
        mesh=mesh,
        compiler_params=pltpu.CompilerParams(needs_layout_passes=False),
        scratch_types=[
            pltpu.VMEM((_CH * 32,), jnp.float32),
            pltpu.VMEM((_CH,), jnp.int32),
            pltpu.VMEM((256,), jnp.float32),
            pltpu.VMEM((16,), jnp.float32),
            pltpu.VMEM((_CH * 16,), jnp.float32),
            pltpu.VMEM((16,), jnp.float32),
            pltpu.VMEM((16,), jnp.float32),
        ],
    )(_sc_chain_body)
    return f(c_flat, sd, w16)


def kernel(src_index, dst_index, timestamp, event_features, labels,
           time_w, time_b, gru_Wih, gru_Whh, gru_bih, gru_bhh, lin_W, lin_b):
    n = _N
    order = jnp.argsort(timestamp)
    s = src_index[order].astype(jnp.int32)
    d = dst_index[order].astype(jnp.int32)
    t = timestamp[order]
    raw = event_features[order]

    rzW = gru_Wih[0:8]
    nW = gru_Wih[8:12]
    Whh8 = gru_Whh[0:8]
    Whh_n = gru_Whh[8:12]
    own = (rzW[:, 0:4] + Whh8).T
    oth = rzW[:, 4:8].T
    B_A = jnp.concatenate([
        jnp.concatenate([own, oth], axis=1),
        jnp.concatenate([oth, own], axis=1),
    ], axis=0)
    z44 = jnp.zeros((4, 4), jnp.float32)
    B_B = jnp.concatenate([
        jnp.concatenate([nW[:, 0:4].T, nW[:, 4:8].T, Whh_n.T, z44], axis=1),
        jnp.concatenate([nW[:, 4:8].T, nW[:, 0:4].T, z44, Whh_n.T], axis=1),
    ], axis=0)
    W16 = jnp.concatenate([B_A, B_B], axis=0)

    z28 = jnp.zeros((2, 8), jnp.float32)
    z48 = jnp.zeros((4, 8), jnp.float32)
    z24 = jnp.zeros((2, 4), jnp.float32)
    z44f = jnp.zeros((4, 4), jnp.float32)
    raw_rz = rzW[:, 8:10].T
    te_rz = rzW[:, 10:14].T
    raw_n = nW[:, 8:10].T
    te_n = nW[:, 10:14].T
    M = jnp.concatenate([
        jnp.concatenate([raw_rz, raw_rz, raw_n, raw_n, z28], axis=1),
        jnp.concatenate([te_rz, z48, te_n, z44f, z48], axis=1),
        jnp.concatenate([z48, te_rz, z44f, te_n, z48], axis=1),
    ], axis=0)
    bias = jnp.concatenate([
        gru_bih[0:8] + gru_bhh[0:8],
        gru_bih[0:8] + gru_bhh[0:8],
        gru_bih[8:12], gru_bih[8:12],
        gru_bhh[8:12], gru_bhh[8:12],
    ])[None, :]

    t2 = t.reshape(_ROWS, _LANES)
    s2 = s.reshape(_ROWS, _LANES)
    d2 = d.reshape(_ROWS, _LANES)
    dts2, dtd2, sd2 = pl.pallas_call(
        _scan_body,
        out_shape=(
            jax.ShapeDtypeStruct((_ROWS, _LANES), jnp.float32),
            jax.ShapeDtypeStruct((_ROWS, _LANES), jnp.float32),
            jax.ShapeDtypeStruct((_ROWS, _LANES), jnp.int32),
        ),
    )(t2, s2, d2)

    C = pl.pallas_call(
        _const_body,
        out_shape=jax.ShapeDtypeStruct((_N, 32), jnp.float32),
    )(dts2.reshape(n, 1), dtd2.reshape(n, 1), raw,
      time_w[None, :], time_b[None, :], M, bias)

    x8out = _sc_chain(C.reshape(n * 32), sd2.reshape(n), W16.reshape(256))

    logits_sorted = pl.pallas_call(
        _logit_body,
        out_shape=jax.ShapeDtypeStruct((_N, 2), jnp.float32),
    )(x8out.reshape(n, 16), raw, lin_W[:, 0:8].T, lin_W[:, 8:10].T,
      lin_b[None, :])

    return jnp.zeros((n, 2), jnp.float32).at[order].set(logits_sorted)


# speedup vs baseline: 277.8358x; 1.0207x over previous
import functools

import jax
import jax.numpy as jnp
from jax import lax
from jax.experimental import pallas as pl
from jax.experimental.pallas import tpu as pltpu
from jax.experimental.pallas import tpu_sc as plsc

_N = 4096
_ROWS = 32
_LANES = 128
_CH = 1024
_NCH = _N // _CH
_MEM = 4
_NODES = 3


def _scan_body(t_ref, s_ref, d_ref, dts_ref, dtd_ref, sd_ref):
    t = t_ref[...]
    s = s_ref[...]
    d = d_ref[...]
    lane = lax.broadcasted_iota(jnp.int32, (_ROWS, _LANES), 1)
    row1 = lax.broadcasted_iota(jnp.int32, (_ROWS, 1), 0)
    prevs = []
    for node in range(_NODES):
        x = jnp.where((s == node) | (d == node), t, 0.0)
        for sh in (1, 2, 4, 8, 16, 32, 64):
            x = jnp.maximum(x, jnp.where(lane >= sh, jnp.roll(x, sh, axis=1), 0.0))
        rt = x[:, _LANES - 1:_LANES]
        e = jnp.where(row1 >= 1, jnp.roll(rt, 1, axis=0), 0.0)
        for sh in (1, 2, 4, 8, 16):
            e = jnp.maximum(e, jnp.where(row1 >= sh, jnp.roll(e, sh, axis=0), 0.0))
        incl = jnp.maximum(x, e)
        excl = jnp.where(lane >= 1, jnp.roll(incl, 1, axis=1), e)
        prevs.append(excl)
    ps = jnp.where(s == 0, prevs[0], jnp.where(s == 1, prevs[1], prevs[2]))
    pd = jnp.where(d == 0, prevs[0], jnp.where(d == 1, prevs[1], prevs[2]))
    dts_ref[...] = t - ps
    dtd_ref[...] = t - pd
    sd_ref[...] = s + d * 4


def _const_body(dts_ref, dtd_ref, raw_ref, w_ref, b_ref, m_ref, bias_ref, c_ref):
    dts = dts_ref[...]
    dtd = dtd_ref[...]
    raw = raw_ref[...]
    w = w_ref[...]
    b = b_ref[...]
    te_s = jnp.cos(dts * w + b)
    te_d = jnp.cos(dtd * w + b)
    acc = jnp.broadcast_to(bias_ref[...], (_N, 32))
    for k in range(2):
        acc = acc + raw[:, k:k + 1] * m_ref[k:k + 1, :]
    for k in range(4):
        acc = acc + te_s[:, k:k + 1] * m_ref[2 + k:3 + k, :]
    for k in range(4):
        acc = acc + te_d[:, k:k + 1] * m_ref[6 + k:7 + k, :]
    c_ref[...] = acc


def _logit_body(x8_ref, raw_ref, lm_ref, lr_ref, lb_ref, o_ref):
    x8 = x8_ref[...]
    raw = raw_ref[...]
    acc = jnp.broadcast_to(lb_ref[...], (_N, 2))
    for k in range(8):
        acc = acc + x8[:, k:k + 1] * lm_ref[k:k + 1, :]
    for k in range(2):
        acc = acc + raw[:, k:k + 1] * lr_ref[k:k + 1, :]
    o_ref[...] = acc


def _vtake(x, idx16):
    return lax.gather(
        x, idx16.reshape(16, 1),
        lax.GatherDimensionNumbers(
            offset_dims=(), collapsed_slice_dims=(0,), start_index_map=(0,)),
        (1,), mode=lax.GatherScatterMode.PROMISE_IN_BOUNDS)


def _sc_chain_body(c_hbm, sd_hbm, w_hbm, out_hbm,
                   cbuf, sdbuf, wbuf, membuf, outbuf):
    cid = lax.axis_index("c")
    sid = lax.axis_index("s")

    @pl.when(jnp.logical_and(cid == 0, sid == 0))
    def _():
        pltpu.sync_copy(w_hbm, wbuf)
        membuf[...] = jnp.zeros((16,), jnp.float32)
        lane = lax.iota(jnp.int32, 16)
        lane_lt4 = lane < 4
        lane_lt8 = lane < 8
        low2 = jnp.bitwise_and(lane, 3)
        perm_hi = jnp.where(lane_lt8, lane + 8, lane)
        bidx = [jnp.full((16,), k, jnp.int32) for k in range(8)]
        wA = [plsc.load_gather(wbuf, [lane + 16 * k]) for k in range(8)]
        wB = [plsc.load_gather(wbuf, [lane + 16 * (8 + k)]) for k in range(8)]

        def _step(j):
            jb = jnp.broadcast_to(j, (16,))
            sdv = plsc.load_gather(sdbuf, [jb])
            sv4 = jnp.bitwise_and(sdv, 3) * 4
            dv4 = lax.shift_right_logical(sdv, 2) * 4
            gidx = jnp.where(lane_lt4, sv4, dv4) + low2
            sidx = jnp.where(sv4 == dv4, 12, sv4)
            widx = jnp.where(lane_lt4, sidx, dv4) + low2
            x8 = plsc.load_gather(membuf, [gidx])
            j32 = j * 32
            yA = plsc.load_gather(cbuf, [lane + j32])
            yB = plsc.load_gather(cbuf, [lane + (j32 + 16)])
            for k in range(8):
                bk = _vtake(x8, bidx[k])
                yA = yA + bk * wA[k]
                yB = yB + bk * wB[k]
            S = 1.0 / (1.0 + jnp.exp(-yA))
            z8 = _vtake(S, perm_hi)
            gh8 = _vtake(yB, perm_hi)
            nin = yB + S * gh8
            e2 = jnp.exp(-2.0 * nin)
            th = (1.0 - e2) / (1.0 + e2)
            new8 = th + z8 * (x8 - th)
            plsc.store_scatter(membuf, [widx], new8, mask=lane_lt8)
            plsc.store_scatter(outbuf, [lane + j * 16], x8)

        @pl.loop(0, _NCH)
        def _chunk(ci):
            pltpu.sync_copy(c_hbm.at[pl.ds(ci * (_CH * 32), _CH * 32)], cbuf)
            pltpu.sync_copy(sd_hbm.at[pl.ds(ci * _CH, _CH)], sdbuf)

            @pl.loop(0, _CH, step=2)
            def _pair(j):
                _step(j)
                _step(j + 1)

            pltpu.sync_copy(outbuf, out_hbm.at[pl.ds(ci * (_CH * 16), _CH * 16)])


def _sc_chain(c_flat, sd, w16):
    mesh = plsc.VectorSubcoreMesh(core_axis_name="c", subcore_axis_name="s")
    f = functools.partial(
        pl.kernel,
        out_type=jax.ShapeDtypeStruct((_N * 16,), jnp.float32),
        mesh=mesh,
        compiler_params=pltpu.CompilerParams(needs_layout_passes=False),
        scratch_types=[
            pltpu.VMEM((_CH * 32,), jnp.float32),
            pltpu.VMEM((_CH,), jnp.int32),
            pltpu.VMEM((256,), jnp.float32),
            pltpu.VMEM((16,), jnp.float32),
            pltpu.VMEM((_CH * 16,), jnp.float32),
        ],
    )(_sc_chain_body)
    return f(c_flat, sd, w16)


def kernel(src_index, dst_index, timestamp, event_features, labels,
           time_w, time_b, gru_Wih, gru_Whh, gru_bih, gru_bhh, lin_W, lin_b):
    n = _N
    order = jnp.argsort(timestamp)
    s = src_index[order].astype(jnp.int32)
    d = dst_index[order].astype(jnp.int32)
    t = timestamp[order]
    raw = event_features[order]

    rzW = gru_Wih[0:8]
    nW = gru_Wih[8:12]
    Whh8 = gru_Whh[0:8]
    Whh_n = gru_Whh[8:12]
    own = (rzW[:, 0:4] + Whh8).T
    oth = rzW[:, 4:8].T
    B_A = jnp.concatenate([
        jnp.concatenate([own, oth], axis=1),
        jnp.concatenate([oth, own], axis=1),
    ], axis=0)
    z44 = jnp.zeros((4, 4), jnp.float32)
    B_B = jnp.concatenate([
        jnp.concatenate([nW[:, 0:4].T, nW[:, 4:8].T, Whh_n.T, z44], axis=1),
        jnp.concatenate([nW[:, 4:8].T, nW[:, 0:4].T, z44, Whh_n.T], axis=1),
    ], axis=0)
    gperm = jnp.asarray([0, 1, 2, 3, 8, 9, 10, 11, 4, 5, 6, 7, 12, 13, 14, 15])
    B_A = B_A[:, gperm]
    W16 = jnp.concatenate([B_A, B_B], axis=0)

    z28 = jnp.zeros((2, 8), jnp.float32)
    z48 = jnp.zeros((4, 8), jnp.float32)
    z24 = jnp.zeros((2, 4), jnp.float32)
    z44f = jnp.zeros((4, 4), jnp.float32)
    raw_rz = rzW[:, 8:10].T
    te_rz = rzW[:, 10:14].T
    raw_n = nW[:, 8:10].T
    te_n = nW[:, 10:14].T
    M = jnp.concatenate([
        jnp.concatenate([raw_rz, raw_rz, raw_n, raw_n, z28], axis=1),
        jnp.concatenate([te_rz, z48, te_n, z44f, z48], axis=1),
        jnp.concatenate([z48, te_rz, z44f, te_n, z48], axis=1),
    ], axis=0)
    M = M.at[:, 0:16].set(M[:, 0:16][:, gperm])
    biasA = (gru_bih[0:8] + gru_bhh[0:8])
    biasA = jnp.concatenate([biasA, biasA])[gperm]
    bias = jnp.concatenate([
        biasA,
        gru_bih[8:12], gru_bih[8:12],
        gru_bhh[8:12], gru_bhh[8:12],
    ])[None, :]

    t2 = t.reshape(_ROWS, _LANES)
    s2 = s.reshape(_ROWS, _LANES)
    d2 = d.reshape(_ROWS, _LANES)
    dts2, dtd2, sd2 = pl.pallas_call(
        _scan_body,
        out_shape=(
            jax.ShapeDtypeStruct((_ROWS, _LANES), jnp.float32),
            jax.ShapeDtypeStruct((_ROWS, _LANES), jnp.float32),
            jax.ShapeDtypeStruct((_ROWS, _LANES), jnp.int32),
        ),
    )(t2, s2, d2)

    C = pl.pallas_call(
        _const_body,
        out_shape=jax.ShapeDtypeStruct((_N, 32), jnp.float32),
    )(dts2.reshape(n, 1), dtd2.reshape(n, 1), raw,
      time_w[None, :], time_b[None, :], M, bias)

    x8out = _sc_chain(C.reshape(n * 32), sd2.reshape(n), W16.reshape(256))

    logits_sorted = pl.pallas_call(
        _logit_body,
        out_shape=jax.ShapeDtypeStruct((_N, 2), jnp.float32),
    )(x8out.reshape(n, 16), raw, lin_W[:, 0:8].T, lin_W[:, 8:10].T,
      lin_b[None, :])

    return jnp.zeros((n, 2), jnp.float32).at[order].set(logits_sorted)


# speedup vs baseline: 315.7779x; 1.1366x over previous
import functools

import jax
import jax.numpy as jnp
from jax import lax
from jax.experimental import pallas as pl
from jax.experimental.pallas import tpu as pltpu
from jax.experimental.pallas import tpu_sc as plsc

_N = 4096
_ROWS = 32
_LANES = 128
_CH = 1024
_NCH = _N // _CH
_MEM = 4
_NODES = 3


def _scan_body(t_ref, s_ref, d_ref, dts_ref, dtd_ref, sd_ref):
    t = t_ref[...]
    s = s_ref[...]
    d = d_ref[...]
    lane = lax.broadcasted_iota(jnp.int32, (_ROWS, _LANES), 1)
    row1 = lax.broadcasted_iota(jnp.int32, (_ROWS, 1), 0)
    prevs = []
    for node in range(_NODES):
        x = jnp.where((s == node) | (d == node), t, 0.0)
        for sh in (1, 2, 4, 8, 16, 32, 64):
            x = jnp.maximum(x, jnp.where(lane >= sh, jnp.roll(x, sh, axis=1), 0.0))
        rt = x[:, _LANES - 1:_LANES]
        e = jnp.where(row1 >= 1, jnp.roll(rt, 1, axis=0), 0.0)
        for sh in (1, 2, 4, 8, 16):
            e = jnp.maximum(e, jnp.where(row1 >= sh, jnp.roll(e, sh, axis=0), 0.0))
        incl = jnp.maximum(x, e)
        excl = jnp.where(lane >= 1, jnp.roll(incl, 1, axis=1), e)
        prevs.append(excl)
    ps = jnp.where(s == 0, prevs[0], jnp.where(s == 1, prevs[1], prevs[2]))
    pd = jnp.where(d == 0, prevs[0], jnp.where(d == 1, prevs[1], prevs[2]))
    dts_ref[...] = t - ps
    dtd_ref[...] = t - pd
    sd_ref[...] = s + d * 4


def _const_body(dts_ref, dtd_ref, raw_ref, w_ref, b_ref, m_ref, bias_ref, c_ref):
    dts = dts_ref[...]
    dtd = dtd_ref[...]
    raw = raw_ref[...]
    w = w_ref[...]
    b = b_ref[...]
    te_s = jnp.cos(dts * w + b)
    te_d = jnp.cos(dtd * w + b)
    acc = jnp.broadcast_to(bias_ref[...], (_N, 32))
    for k in range(2):
        acc = acc + raw[:, k:k + 1] * m_ref[k:k + 1, :]
    for k in range(4):
        acc = acc + te_s[:, k:k + 1] * m_ref[2 + k:3 + k, :]
    for k in range(4):
        acc = acc + te_d[:, k:k + 1] * m_ref[6 + k:7 + k, :]
    c_ref[...] = acc


def _logit_body(x8_ref, raw_ref, lm_ref, lr_ref, lb_ref, o_ref):
    x8 = x8_ref[...]
    raw = raw_ref[...]
    acc = jnp.broadcast_to(lb_ref[...], (_N, 2))
    for k in range(8):
        acc = acc + x8[:, k:k + 1] * lm_ref[k:k + 1, :]
    for k in range(2):
        acc = acc + raw[:, k:k + 1] * lr_ref[k:k + 1, :]
    o_ref[...] = acc


def _vtake(x, idx16):
    return lax.gather(
        x, idx16.reshape(16, 1),
        lax.GatherDimensionNumbers(
            offset_dims=(), collapsed_slice_dims=(0,), start_index_map=(0,)),
        (1,), mode=lax.GatherScatterMode.PROMISE_IN_BOUNDS)


def _sc_chain_body(c_hbm, sd_hbm, w_hbm, out_hbm,
                   cbuf, sdbuf, wbuf, outbuf):
    cid = lax.axis_index("c")
    sid = lax.axis_index("s")

    @pl.when(jnp.logical_and(cid == 0, sid == 0))
    def _():
        pltpu.sync_copy(w_hbm, wbuf)
        lane = lax.iota(jnp.int32, 16)
        lane_lt4 = lane < 4
        low2 = jnp.bitwise_and(lane, 3)
        low2p4 = low2 + 4
        rowid = lax.shift_right_logical(lane, 2)
        perm_hi = jnp.where(lane < 8, lane + 8, lane)
        bidx = [jnp.full((16,), k, jnp.int32) for k in range(8)]
        wA = [plsc.load_gather(wbuf, [lane + 16 * k]) for k in range(8)]
        wB = [plsc.load_gather(wbuf, [lane + 16 * (8 + k)]) for k in range(8)]

        def _step(j, mem16):
            jb = jnp.broadcast_to(j, (16,))
            sdv = plsc.load_gather(sdbuf, [jb])
            sv = jnp.bitwise_and(sdv, 3)
            dv = lax.shift_right_logical(sdv, 2)
            gidx = jnp.where(lane_lt4, sv, dv) * 4 + low2
            x8 = _vtake(mem16, gidx)
            j32 = j * 32
            yA = plsc.load_gather(cbuf, [lane + j32])
            yB = plsc.load_gather(cbuf, [lane + (j32 + 16)])
            for k in range(8):
                bk = _vtake(x8, bidx[k])
                yA = yA + bk * wA[k]
                yB = yB + bk * wB[k]
            S = 1.0 / (1.0 + jnp.exp(-yA))
            z8 = _vtake(S, perm_hi)
            gh8 = _vtake(yB, perm_hi)
            nin = yB + S * gh8
            e2 = jnp.exp(-2.0 * nin)
            th = (1.0 - e2) / (1.0 + e2)
            new8 = th + z8 * (x8 - th)
            is_d = rowid == dv
            touched = jnp.logical_or(is_d, rowid == sv)
            upd = _vtake(new8, jnp.where(is_d, low2p4, low2))
            plsc.store_scatter(outbuf, [lane + j * 16], x8)
            return jnp.where(touched, upd, mem16)

        def _chunk(ci, mem16):
            pltpu.sync_copy(c_hbm.at[pl.ds(ci * (_CH * 32), _CH * 32)], cbuf)
            pltpu.sync_copy(sd_hbm.at[pl.ds(ci * _CH, _CH)], sdbuf)

            def _pair(jj, m):
                m = _step(jj * 2, m)
                return _step(jj * 2 + 1, m)

            mem16 = lax.fori_loop(0, _CH // 2, _pair, mem16)
            pltpu.sync_copy(outbuf, out_hbm.at[pl.ds(ci * (_CH * 16), _CH * 16)])
            return mem16

        lax.fori_loop(0, _NCH, _chunk, jnp.zeros((16,), jnp.float32))


def _sc_chain(c_flat, sd, w16):
    mesh = plsc.VectorSubcoreMesh(core_axis_name="c", subcore_axis_name="s")
    f = functools.partial(
        pl.kernel,
        out_type=jax.ShapeDtypeStruct((_N * 16,), jnp.float32),
        mesh=mesh,
        compiler_params=pltpu.CompilerParams(needs_layout_passes=False),
        scratch_types=[
            pltpu.VMEM((_CH * 32,), jnp.float32),
            pltpu.VMEM((_CH,), jnp.int32),
            pltpu.VMEM((256,), jnp.float32),
            pltpu.VMEM((_CH * 16,), jnp.float32),
        ],
    )(_sc_chain_body)
    return f(c_flat, sd, w16)


def kernel(src_index, dst_index, timestamp, event_features, labels,
           time_w, time_b, gru_Wih, gru_Whh, gru_bih, gru_bhh, lin_W, lin_b):
    n = _N
    order = jnp.argsort(timestamp)
    s = src_index[order].astype(jnp.int32)
    d = dst_index[order].astype(jnp.int32)
    t = timestamp[order]
    raw = event_features[order]

    rzW = gru_Wih[0:8]
    nW = gru_Wih[8:12]
    Whh8 = gru_Whh[0:8]
    Whh_n = gru_Whh[8:12]
    own = (rzW[:, 0:4] + Whh8).T
    oth = rzW[:, 4:8].T
    B_A = jnp.concatenate([
        jnp.concatenate([own, oth], axis=1),
        jnp.concatenate([oth, own], axis=1),
    ], axis=0)
    z44 = jnp.zeros((4, 4), jnp.float32)
    B_B = jnp.concatenate([
        jnp.concatenate([nW[:, 0:4].T, nW[:, 4:8].T, Whh_n.T, z44], axis=1),
        jnp.concatenate([nW[:, 4:8].T, nW[:, 0:4].T, z44, Whh_n.T], axis=1),
    ], axis=0)
    gperm = jnp.asarray([0, 1, 2, 3, 8, 9, 10, 11, 4, 5, 6, 7, 12, 13, 14, 15])
    B_A = B_A[:, gperm]
    W16 = jnp.concatenate([B_A, B_B], axis=0)

    z28 = jnp.zeros((2, 8), jnp.float32)
    z48 = jnp.zeros((4, 8), jnp.float32)
    z24 = jnp.zeros((2, 4), jnp.float32)
    z44f = jnp.zeros((4, 4), jnp.float32)
    raw_rz = rzW[:, 8:10].T
    te_rz = rzW[:, 10:14].T
    raw_n = nW[:, 8:10].T
    te_n = nW[:, 10:14].T
    M = jnp.concatenate([
        jnp.concatenate([raw_rz, raw_rz, raw_n, raw_n, z28], axis=1),
        jnp.concatenate([te_rz, z48, te_n, z44f, z48], axis=1),
        jnp.concatenate([z48, te_rz, z44f, te_n, z48], axis=1),
    ], axis=0)
    M = M.at[:, 0:16].set(M[:, 0:16][:, gperm])
    biasA = (gru_bih[0:8] + gru_bhh[0:8])
    biasA = jnp.concatenate([biasA, biasA])[gperm]
    bias = jnp.concatenate([
        biasA,
        gru_bih[8:12], gru_bih[8:12],
        gru_bhh[8:12], gru_bhh[8:12],
    ])[None, :]

    t2 = t.reshape(_ROWS, _LANES)
    s2 = s.reshape(_ROWS, _LANES)
    d2 = d.reshape(_ROWS, _LANES)
    dts2, dtd2, sd2 = pl.pallas_call(
        _scan_body,
        out_shape=(
            jax.ShapeDtypeStruct((_ROWS, _LANES), jnp.float32),
            jax.ShapeDtypeStruct((_ROWS, _LANES), jnp.float32),
            jax.ShapeDtypeStruct((_ROWS, _LANES), jnp.int32),
        ),
    )(t2, s2, d2)

    C = pl.pallas_call(
        _const_body,
        out_shape=jax.ShapeDtypeStruct((_N, 32), jnp.float32),
    )(dts2.reshape(n, 1), dtd2.reshape(n, 1), raw,
      time_w[None, :], time_b[None, :], M, bias)

    x8out = _sc_chain(C.reshape(n * 32), sd2.reshape(n), W16.reshape(256))

    logits_sorted = pl.pallas_call(
        _logit_body,
        out_shape=jax.ShapeDtypeStruct((_N, 2), jnp.float32),
    )(x8out.reshape(n, 16), raw, lin_W[:, 0:8].T, lin_W[:, 8:10].T,
      lin_b[None, :])

    return jnp.zeros((n, 2), jnp.float32).at[order].set(logits_sorted)
